# Initial kernel scaffold; baseline (speedup 1.0000x reference)
import jax
import jax.numpy as jnp
from jax.experimental import pallas as pl


def kernel(bf1, bf2, ba, bd, adj, W1, b1, a1, W2, b2, a2, num_hop, sparse):
    raise NotImplementedError("write your pallas kernel here")


# speedup vs baseline: 1.5229x; 1.5229x over previous
import functools

import jax
import jax.numpy as jnp
from jax.experimental import pallas as pl

_BM = 400


def _sf_body(bf1_ref, bf2_ref, w1_ref, w2_ref, sf1_ref, sf2_ref):
    sf1_ref[...] = jax.lax.dot_general(
        bf1_ref[...], w1_ref[...], (((1,), (1,)), ((), ())),
        preferred_element_type=jnp.float32)
    sf2_ref[...] = jax.lax.dot_general(
        bf2_ref[...], w2_ref[...], (((1,), (1,)), ((), ())),
        preferred_element_type=jnp.float32)


def _gcn_body(adj_ref, sf_ref, b_ref, a_ref, out_ref, *, normalize):
    acc = jax.lax.dot_general(
        adj_ref[...].astype(jnp.bfloat16), sf_ref[...].astype(jnp.bfloat16),
        (((1,), (0,)), ((), ())), preferred_element_type=jnp.float32)
    out = acc + b_ref[...]
    a = a_ref[0, 0]
    out = jnp.where(out >= 0, out, a * out)
    if normalize:
        nrm = jnp.sqrt(jnp.sum(out * out, axis=1, keepdims=True))
        out = out / jnp.maximum(nrm, 1e-12)
    out_ref[...] = out


def _hop_body(adj_ref, x_ref, out_ref):
    out_ref[...] = jax.lax.dot_general(
        adj_ref[...].astype(jnp.bfloat16), x_ref[...].astype(jnp.bfloat16),
        (((1,), (0,)), ((), ())), preferred_element_type=jnp.float32)


def _loss_body(za_blk_ref, nb_blk_ref, za_all_ref, out_ref):
    z = za_blk_ref[...]
    nb = nb_blk_ref[...]
    nrm = jnp.sqrt(jnp.sum(nb * nb, axis=1, keepdims=True))
    znb = nb / jnp.maximum(nrm, 1e-12)
    inter = jnp.sum(z * znb, axis=1)
    diag = jnp.sum(z * z, axis=1)
    sim = jax.lax.dot_general(
        z.astype(jnp.bfloat16), za_all_ref[...].astype(jnp.bfloat16),
        (((1,), (1,)), ((), ())), preferred_element_type=jnp.float32)
    s = jnp.sum(jnp.exp(sim), axis=1)
    li = -inter + jnp.log(s - jnp.exp(diag))

    @pl.when(pl.program_id(0) == 0)
    def _():
        out_ref[...] = jnp.zeros((1, 1), jnp.float32)

    out_ref[...] += jnp.sum(li).reshape(1, 1)


def kernel(bf1, bf2, ba, bd, adj, W1, b1, a1, W2, b2, a2, num_hop, sparse):
    n = ba.shape[-1]
    nin = bf1.shape[-1]
    nh = W1.shape[0]
    bm = _BM
    nblk = n // bm

    x1 = bf1.reshape(n, nin)
    x2 = bf2.reshape(n, nin)
    A = ba.reshape(n, n)
    D = bd.reshape(n, n)
    G = adj.reshape(n, n)
    b1r = b1.reshape(1, nh)
    b2r = b2.reshape(1, nh)
    a1r = a1.reshape(1, 1)
    a2r = a2.reshape(1, 1)

    sf1, sf2 = pl.pallas_call(
        _sf_body,
        out_shape=[jax.ShapeDtypeStruct((n, nh), jnp.float32)] * 2,
    )(x1, x2, W1, W2)

    adj_spec = pl.BlockSpec((bm, n), lambda i: (i, 0))
    sf_spec = pl.BlockSpec((n, nh), lambda i: (0, 0))
    row_spec = pl.BlockSpec((bm, nh), lambda i: (i, 0))
    vec_spec = pl.BlockSpec((1, nh), lambda i: (0, 0))
    scl_spec = pl.BlockSpec((1, 1), lambda i: (0, 0))

    def gcn(adjmat, sf, b, a, normalize):
        return pl.pallas_call(
            functools.partial(_gcn_body, normalize=normalize),
            grid=(nblk,),
            in_specs=[adj_spec, sf_spec, vec_spec, scl_spec],
            out_specs=row_spec,
            out_shape=jax.ShapeDtypeStruct((n, nh), jnp.float32),
        )(adjmat, sf, b, a)

    za = gcn(A, sf1, b1r, a1r, normalize=True)
    b_emb = gcn(D, sf2, b2r, a2r, normalize=False)

    def hop(_, x):
        return pl.pallas_call(
            _hop_body,
            grid=(nblk,),
            in_specs=[adj_spec, sf_spec],
            out_specs=row_spec,
            out_shape=jax.ShapeDtypeStruct((n, nh), jnp.float32),
        )(G, x)

    nb = jax.lax.fori_loop(0, num_hop, hop, b_emb)

    loss_sum = pl.pallas_call(
        _loss_body,
        grid=(nblk,),
        in_specs=[row_spec, row_spec, sf_spec],
        out_specs=pl.BlockSpec((1, 1), lambda i: (0, 0)),
        out_shape=jax.ShapeDtypeStruct((1, 1), jnp.float32),
    )(za, nb, za)

    return loss_sum[0, 0] / n


# speedup vs baseline: 1.6530x; 1.0855x over previous
import functools

import jax
import jax.numpy as jnp
from jax.experimental import pallas as pl

_BM = 400


def _sf_body(bf1_ref, bf2_ref, w1_ref, w2_ref, sf1_ref, sf2_ref):
    sf1_ref[...] = jax.lax.dot_general(
        bf1_ref[...], w1_ref[...], (((1,), (1,)), ((), ())),
        preferred_element_type=jnp.float32)
    sf2_ref[...] = jax.lax.dot_general(
        bf2_ref[...], w2_ref[...], (((1,), (1,)), ((), ())),
        preferred_element_type=jnp.float32)


def _gcn_a_body(adj_ref, sf_ref, b_ref, a_ref, out_ref):
    acc = jax.lax.dot_general(
        adj_ref[...].astype(jnp.bfloat16), sf_ref[...].astype(jnp.bfloat16),
        (((1,), (0,)), ((), ())), preferred_element_type=jnp.float32)
    out = acc + b_ref[...]
    a = a_ref[0, 0]
    out = jnp.where(out >= 0, out, a * out)
    nrm = jnp.sqrt(jnp.sum(out * out, axis=1, keepdims=True))
    out_ref[...] = out / jnp.maximum(nrm, 1e-12)


def _gcn_b_body(adj_ref, sf_ref, b_ref, a_ref, za_ref, out_ref, s_ref):
    acc = jax.lax.dot_general(
        adj_ref[...].astype(jnp.bfloat16), sf_ref[...].astype(jnp.bfloat16),
        (((1,), (0,)), ((), ())), preferred_element_type=jnp.float32)
    out = acc + b_ref[...]
    a = a_ref[0, 0]
    out_ref[...] = jnp.where(out >= 0, out, a * out)

    i = pl.program_id(0)
    bm = out_ref.shape[0]
    z = za_ref[pl.ds(i * bm, bm), :]
    sim = jax.lax.dot_general(
        z.astype(jnp.bfloat16), za_ref[...].astype(jnp.bfloat16),
        (((1,), (1,)), ((), ())), preferred_element_type=jnp.float32)
    s_ref[...] = jnp.sum(jnp.exp(sim), axis=1, keepdims=True)


def _hop_body(adj_ref, x_ref, out_ref):
    out_ref[...] = jax.lax.dot_general(
        adj_ref[...].astype(jnp.bfloat16), x_ref[...].astype(jnp.bfloat16),
        (((1,), (0,)), ((), ())), preferred_element_type=jnp.float32)


def _loss_body(za_ref, nb_ref, s_ref, out_ref):
    z = za_ref[...]
    nb = nb_ref[...]
    nrm = jnp.sqrt(jnp.sum(nb * nb, axis=1, keepdims=True))
    znb = nb / jnp.maximum(nrm, 1e-12)
    inter = jnp.sum(z * znb, axis=1)
    diag = jnp.sum(z * z, axis=1)
    li = -inter + jnp.log(s_ref[...][:, 0] - jnp.exp(diag))
    out_ref[...] = jnp.mean(li).reshape(1, 1)


def kernel(bf1, bf2, ba, bd, adj, W1, b1, a1, W2, b2, a2, num_hop, sparse):
    n = ba.shape[-1]
    nin = bf1.shape[-1]
    nh = W1.shape[0]
    bm = _BM
    nblk = n // bm

    x1 = bf1.reshape(n, nin)
    x2 = bf2.reshape(n, nin)
    A = ba.reshape(n, n)
    D = bd.reshape(n, n)
    G = adj.reshape(n, n)
    b1r = b1.reshape(1, nh)
    b2r = b2.reshape(1, nh)
    a1r = a1.reshape(1, 1)
    a2r = a2.reshape(1, 1)

    sf1, sf2 = pl.pallas_call(
        _sf_body,
        out_shape=[jax.ShapeDtypeStruct((n, nh), jnp.float32)] * 2,
    )(x1, x2, W1, W2)

    adj_spec = pl.BlockSpec((bm, n), lambda i: (i, 0))
    full_spec = pl.BlockSpec((n, nh), lambda i: (0, 0))
    row_spec = pl.BlockSpec((bm, nh), lambda i: (i, 0))
    vec_spec = pl.BlockSpec((1, nh), lambda i: (0, 0))
    scl_spec = pl.BlockSpec((1, 1), lambda i: (0, 0))

    za = pl.pallas_call(
        _gcn_a_body,
        grid=(nblk,),
        in_specs=[adj_spec, full_spec, vec_spec, scl_spec],
        out_specs=row_spec,
        out_shape=jax.ShapeDtypeStruct((n, nh), jnp.float32),
    )(A, sf1, b1r, a1r)

    b_emb, s_intra = pl.pallas_call(
        _gcn_b_body,
        grid=(nblk,),
        in_specs=[adj_spec, full_spec, vec_spec, scl_spec, full_spec],
        out_specs=[row_spec, pl.BlockSpec((bm, 1), lambda i: (i, 0))],
        out_shape=[jax.ShapeDtypeStruct((n, nh), jnp.float32),
                   jax.ShapeDtypeStruct((n, 1), jnp.float32)],
    )(D, sf2, b2r, a2r, za)

    def hop(_, x):
        return pl.pallas_call(
            _hop_body,
            grid=(nblk,),
            in_specs=[adj_spec, full_spec],
            out_specs=row_spec,
            out_shape=jax.ShapeDtypeStruct((n, nh), jnp.float32),
        )(G, x)

    nb = jax.lax.fori_loop(0, num_hop, hop, b_emb)

    loss = pl.pallas_call(
        _loss_body,
        out_shape=jax.ShapeDtypeStruct((1, 1), jnp.float32),
    )(za, nb, s_intra)

    return loss[0, 0]
